# baseline (speedup 1.0000x reference)
import jax
import jax.numpy as jnp
from jax import lax
from jax.experimental import pallas as pl
from jax.experimental.pallas import tpu as pltpu
from jax.experimental.pallas import tpu_sc as plsc

_TEMPERATURE = 0.07
_MARGIN = 0.3
_B = 1024
_D = 64
_RB = 128
_RB2 = 8
_IB = 128
_NC = 2
_NS = 16
_NW = _NC * _NS
_L = 16


def _stats_body(a_ref, c_ref, pm_ref, ks_ref, kn_ref, loss_ref, acc_ref,
                cnt_ref):
    step = pl.program_id(0)
    an = a_ref[...]
    cn = c_ref[...]
    pm = pm_ref[...]
    sim = lax.dot_general(an, cn, (((1,), (1,)), ((), ())),
                          preferred_element_type=jnp.float32)
    cnt = jnp.sum(pm, axis=1, keepdims=True)
    pos = jnp.sum(sim * pm, axis=1, keepdims=True) / jnp.maximum(cnt, 1.0)
    negm = pm == 0.0
    semi = jnp.logical_and(sim > pos - _MARGIN, negm)
    neg_inf = jnp.float32(-jnp.inf)
    ks_ref[...] = jnp.where(semi, sim, neg_inf)
    kn_ref[...] = jnp.where(negm, sim, neg_inf)
    row_max = jnp.max(sim, axis=1, keepdims=True)
    m = jnp.maximum(row_max, pos)
    se = (jnp.sum(jnp.exp((sim - m) / _TEMPERATURE), axis=1, keepdims=True)
          + jnp.exp((pos - m) / _TEMPERATURE))
    loss_rows = m / _TEMPERATURE + jnp.log(se) - pos / _TEMPERATURE
    acc_rows = (pos >= row_max).astype(jnp.float32)
    lsum = jnp.sum(loss_rows)
    asum = jnp.sum(acc_rows)
    ssum = jnp.sum(semi.astype(jnp.float32))

    @pl.when(step == 0)
    def _():
        loss_ref[0, 0] = lsum
        acc_ref[0, 0] = asum
        cnt_ref[0, 0] = ssum

    @pl.when(step != 0)
    def _():
        loss_ref[0, 0] += lsum
        acc_ref[0, 0] += asum
        cnt_ref[0, 0] += ssum


def _mine_stats(anchors_n, candidates_n, pm_f):
    return pl.pallas_call(
        _stats_body,
        grid=(_B // _RB,),
        in_specs=[
            pl.BlockSpec((_RB, _D), lambda i: (i, 0)),
            pl.BlockSpec((_B, _D), lambda i: (0, 0)),
            pl.BlockSpec((_RB, _B), lambda i: (i, 0)),
        ],
        out_specs=[
            pl.BlockSpec((_RB, _B), lambda i: (i, 0)),
            pl.BlockSpec((_RB, _B), lambda i: (i, 0)),
            pl.BlockSpec(memory_space=pltpu.SMEM),
            pl.BlockSpec(memory_space=pltpu.SMEM),
            pl.BlockSpec(memory_space=pltpu.SMEM),
        ],
        out_shape=[
            jax.ShapeDtypeStruct((_B, _B), jnp.float32),
            jax.ShapeDtypeStruct((_B, _B), jnp.float32),
            jax.ShapeDtypeStruct((1, 1), jnp.float32),
            jax.ShapeDtypeStruct((1, 1), jnp.float32),
            jax.ShapeDtypeStruct((1, 1), jnp.float32),
        ],
    )(anchors_n, candidates_n, pm_f)


def _prep_body(cnt_ref, ks_ref, kn_ref, key_ref, dest_ref):
    use_semi = cnt_ref[0, 0] > 0.0
    key = jnp.where(use_semi, ks_ref[...], kn_ref[...])
    key_ref[...] = key
    one = jnp.float32(1.0)
    zero = jnp.float32(0.0)
    neg_inf = jnp.float32(-jnp.inf)
    mt = jnp.where(key == neg_inf, zero, one)
    t0 = lax.broadcasted_iota(jnp.int32, (_B, _B), 0)
    t1 = lax.broadcasted_iota(jnp.int32, (_B, _B), 1)
    tri = jnp.where(t0 <= t1, one, zero)
    pref = lax.dot_general(mt, tri, (((1,), (0,)), ((), ())),
                           preferred_element_type=jnp.float32)
    cb = pref[:, _B - 1:_B]
    jj = lax.broadcasted_iota(jnp.int32, (_RB, _B), 1).astype(jnp.float32)
    dest_f = jnp.where(mt > 0.5, pref - one, cb + jj - pref)
    dest_ref[...] = dest_f.astype(jnp.int32)


def _prep(cnt_s, ks, kn):
    return pl.pallas_call(
        _prep_body,
        grid=(_B // _RB,),
        in_specs=[
            pl.BlockSpec(memory_space=pltpu.SMEM),
            pl.BlockSpec((_RB, _B), lambda i: (i, 0)),
            pl.BlockSpec((_RB, _B), lambda i: (i, 0)),
        ],
        out_specs=[
            pl.BlockSpec((_RB, _B), lambda i: (i, 0)),
            pl.BlockSpec((_RB, _B), lambda i: (i, 0)),
        ],
        out_shape=[
            jax.ShapeDtypeStruct((_B, _B), jnp.float32),
            jax.ShapeDtypeStruct((_B, _B), jnp.int32),
        ],
    )(cnt_s, ks, kn)


def _compact_body(key_hbm, dest_hbm, kc_hbm, oc_hbm, kv, dv, kcv, ocv):
    wid = lax.axis_index("s") * _NC + lax.axis_index("c")
    rows_per = _B // _NW

    def row_step(r, carry):
        row = wid * rows_per + r
        pltpu.sync_copy(key_hbm.at[row], kv)
        pltpu.sync_copy(dest_hbm.at[row], dv)

        def chunk(k, c2):
            d = dv[pl.ds(k * _L, _L)]
            v = kv[pl.ds(k * _L, _L)]
            ids = lax.broadcasted_iota(jnp.int32, (_L,), 0) + k * _L
            plsc.store_scatter(kcv, [d], v)
            plsc.store_scatter(ocv, [d], ids)
            return c2

        lax.fori_loop(0, _B // _L, chunk, 0)
        pltpu.sync_copy(kcv, kc_hbm.at[row])
        pltpu.sync_copy(ocv, oc_hbm.at[row])
        return carry

    lax.fori_loop(0, rows_per, row_step, 0)


def _compact(keysel, dest):
    f = pl.kernel(
        _compact_body,
        mesh=plsc.VectorSubcoreMesh(core_axis_name="c", subcore_axis_name="s"),
        out_type=[
            jax.ShapeDtypeStruct((_B, _B), jnp.float32),
            jax.ShapeDtypeStruct((_B, _B), jnp.int32),
        ],
        scratch_types=[
            pltpu.VMEM((_B,), jnp.float32),
            pltpu.VMEM((_B,), jnp.int32),
            pltpu.VMEM((_B,), jnp.float32),
            pltpu.VMEM((_B,), jnp.int32),
        ],
        compiler_params=pltpu.CompilerParams(needs_layout_passes=False),
    )
    return f(keysel, dest)


def _rank_body(cmax_ref, kc_ref, out_ref, acc_ref):
    b = pl.program_id(0)
    cmax = cmax_ref[b, 0]
    kj = kc_ref[...]
    one = jnp.float32(1.0)
    zero = jnp.float32(0.0)
    ii = lax.broadcasted_iota(jnp.int32, (_IB, _IB), 0)
    jj = lax.broadcasted_iota(jnp.int32, (_IB, _IB), 1)
    tie_f = jnp.where(ii < jj, one, zero)[None, :, :]
    acc_ref[...] = jnp.zeros((_RB2, _B), jnp.float32)
    nb = _B // _IB
    for ib in range(nb):
        ilo = ib * _IB
        hi = ilo + _IB
        ki3 = kj[:, ilo:hi, None]

        @pl.when(ilo < cmax)
        def _(ki3=ki3, ilo=ilo, hi=hi):
            pieces = []
            if ilo > 0:
                left = kj[:, None, :ilo]
                pieces.append(
                    jnp.sum(jnp.where(ki3 > left, one, zero), axis=1))
            diag = kj[:, None, ilo:hi]
            d = (jnp.where(ki3 > diag, one, zero)
                 + tie_f * jnp.where(ki3 == diag, one, zero))
            pieces.append(jnp.sum(d, axis=1))
            if hi < _B:
                right = kj[:, None, hi:]
                pieces.append(
                    jnp.sum(jnp.where(ki3 >= right, one, zero), axis=1))
            acc_ref[...] += jnp.concatenate(pieces, axis=1)

    jrow = lax.broadcasted_iota(jnp.int32, (_RB2, _IB), 1)
    for jb in range(nb):
        jlo = jb * _IB
        blk = acc_ref[:, jlo:jlo + _IB].astype(jnp.int32)
        out_ref[:, jlo:jlo + _IB] = jnp.where(jlo < cmax, blk, jrow + jlo)


def _rank(cmax, kc):
    return pl.pallas_call(
        _rank_body,
        grid=(_B // _RB2,),
        in_specs=[
            pl.BlockSpec(memory_space=pltpu.SMEM),
            pl.BlockSpec((_RB2, _B), lambda b: (b, 0)),
        ],
        out_specs=pl.BlockSpec((_RB2, _B), lambda b: (b, 0)),
        out_shape=jax.ShapeDtypeStruct((_B, _B), jnp.int32),
        scratch_shapes=[pltpu.VMEM((_RB2, _B), jnp.float32)],
    )(cmax, kc)


def _invert_body(rank_hbm, oc_hbm, out_hbm, rv, ov, inv_v):
    wid = lax.axis_index("s") * _NC + lax.axis_index("c")
    rows_per = _B // _NW

    def row_step(r, carry):
        row = wid * rows_per + r
        pltpu.sync_copy(rank_hbm.at[row], rv)
        pltpu.sync_copy(oc_hbm.at[row], ov)

        def chunk(k, c2):
            idx = rv[pl.ds(k * _L, _L)]
            vals = ov[pl.ds(k * _L, _L)]
            plsc.store_scatter(inv_v, [idx], vals)
            return c2

        lax.fori_loop(0, _B // _L, chunk, 0)
        pltpu.sync_copy(inv_v, out_hbm.at[row])
        return carry

    lax.fori_loop(0, rows_per, row_step, 0)


def _invert(rank_c, oc):
    f = pl.kernel(
        _invert_body,
        mesh=plsc.VectorSubcoreMesh(core_axis_name="c", subcore_axis_name="s"),
        out_type=jax.ShapeDtypeStruct((_B, _B), jnp.int32),
        scratch_types=[
            pltpu.VMEM((_B,), jnp.int32),
            pltpu.VMEM((_B,), jnp.int32),
            pltpu.VMEM((_B,), jnp.int32),
        ],
        compiler_params=pltpu.CompilerParams(needs_layout_passes=False),
    )
    return f(rank_c, oc)


def _l2n(x):
    return x / jnp.maximum(jnp.linalg.norm(x, axis=1, keepdims=True), 1e-12)


def kernel(anchors, candidates, positive_mask):
    pm_f = positive_mask.astype(jnp.float32)
    ks, kn, loss_s, acc_s, cnt_s = _mine_stats(_l2n(anchors),
                                               _l2n(candidates), pm_f)
    keysel, dest = _prep(cnt_s, ks, kn)
    kc, oc = _compact(keysel, dest)
    fin = jnp.sum(keysel > -jnp.inf, axis=1).astype(jnp.int32)
    cmax = fin.reshape(_B // _RB2, _RB2).max(axis=1).reshape(-1, 1)
    rank_c = _rank(cmax, kc)
    hard_indices = _invert(rank_c, oc)
    loss = loss_s[0, 0] / _B
    accuracy = acc_s[0, 0] / _B
    return loss, accuracy, hard_indices


# speedup vs baseline: 1.2018x; 1.2018x over previous
import jax
import jax.numpy as jnp
from jax import lax
from jax.experimental import pallas as pl
from jax.experimental.pallas import tpu as pltpu
from jax.experimental.pallas import tpu_sc as plsc

_TEMPERATURE = 0.07
_MARGIN = 0.3
_B = 1024
_D = 64
_RB = 128
_RB2 = 16
_IB = 128
_NC = 2
_NS = 16
_NW = _NC * _NS
_L = 16


def _stats_body(a_ref, c_ref, pm_ref, ks_ref, kn_ref, loss_ref, acc_ref,
                cnt_ref):
    step = pl.program_id(0)
    an = a_ref[...]
    cn = c_ref[...]
    pm = pm_ref[...]
    sim = lax.dot_general(an, cn, (((1,), (1,)), ((), ())),
                          preferred_element_type=jnp.float32)
    cnt = jnp.sum(pm, axis=1, keepdims=True)
    pos = jnp.sum(sim * pm, axis=1, keepdims=True) / jnp.maximum(cnt, 1.0)
    negm = pm == 0.0
    semi = jnp.logical_and(sim > pos - _MARGIN, negm)
    neg_inf = jnp.float32(-jnp.inf)
    ks_ref[...] = jnp.where(semi, sim, neg_inf)
    kn_ref[...] = jnp.where(negm, sim, neg_inf)
    row_max = jnp.max(sim, axis=1, keepdims=True)
    m = jnp.maximum(row_max, pos)
    se = (jnp.sum(jnp.exp((sim - m) / _TEMPERATURE), axis=1, keepdims=True)
          + jnp.exp((pos - m) / _TEMPERATURE))
    loss_rows = m / _TEMPERATURE + jnp.log(se) - pos / _TEMPERATURE
    acc_rows = (pos >= row_max).astype(jnp.float32)
    lsum = jnp.sum(loss_rows)
    asum = jnp.sum(acc_rows)
    ssum = jnp.sum(semi.astype(jnp.float32))

    @pl.when(step == 0)
    def _():
        loss_ref[0, 0] = lsum
        acc_ref[0, 0] = asum
        cnt_ref[0, 0] = ssum

    @pl.when(step != 0)
    def _():
        loss_ref[0, 0] += lsum
        acc_ref[0, 0] += asum
        cnt_ref[0, 0] += ssum


def _mine_stats(anchors, candidates, pm_f):
    return pl.pallas_call(
        _stats_body,
        grid=(_B // _RB,),
        in_specs=[
            pl.BlockSpec((_RB, _D), lambda i: (i, 0)),
            pl.BlockSpec((_B, _D), lambda i: (0, 0)),
            pl.BlockSpec((_RB, _B), lambda i: (i, 0)),
        ],
        out_specs=[
            pl.BlockSpec((_RB, _B), lambda i: (i, 0)),
            pl.BlockSpec((_RB, _B), lambda i: (i, 0)),
            pl.BlockSpec(memory_space=pltpu.SMEM),
            pl.BlockSpec(memory_space=pltpu.SMEM),
            pl.BlockSpec(memory_space=pltpu.SMEM),
        ],
        out_shape=[
            jax.ShapeDtypeStruct((_B, _B), jnp.float32),
            jax.ShapeDtypeStruct((_B, _B), jnp.float32),
            jax.ShapeDtypeStruct((1, 1), jnp.float32),
            jax.ShapeDtypeStruct((1, 1), jnp.float32),
            jax.ShapeDtypeStruct((1, 1), jnp.float32),
        ],
    )(anchors, candidates, pm_f)


def _rank_body(cnt_ref, ksj_ref, knj_ref, out_ref):
    use_semi = cnt_ref[0, 0] > 0.0
    kj = jnp.where(use_semi, ksj_ref[...], knj_ref[...])
    one = jnp.float32(1.0)
    zero = jnp.float32(0.0)
    ii = lax.broadcasted_iota(jnp.int32, (_IB, _IB), 0)
    jj = lax.broadcasted_iota(jnp.int32, (_IB, _IB), 1)
    tie_f = jnp.where(ii < jj, one, zero)[None, :, :]
    tot = None
    for ib in range(_B // _IB):
        lo = ib * _IB
        hi = lo + _IB
        ki3 = kj[:, lo:hi, None]
        pieces = []
        if lo > 0:
            left = kj[:, None, :lo]
            pieces.append(jnp.sum(jnp.where(ki3 > left, one, zero), axis=1))
        diag = kj[:, None, lo:hi]
        d = (jnp.where(ki3 > diag, one, zero)
             + tie_f * jnp.where(ki3 == diag, one, zero))
        pieces.append(jnp.sum(d, axis=1))
        if hi < _B:
            right = kj[:, None, hi:]
            pieces.append(jnp.sum(jnp.where(ki3 >= right, one, zero), axis=1))
        contrib = jnp.concatenate(pieces, axis=1)
        tot = contrib if tot is None else tot + contrib
    out_ref[...] = tot.astype(jnp.int32)


def _rank(cnt_s, ks, kn):
    return pl.pallas_call(
        _rank_body,
        grid=(_B // _RB2,),
        in_specs=[
            pl.BlockSpec(memory_space=pltpu.SMEM),
            pl.BlockSpec((_RB2, _B), lambda b: (b, 0)),
            pl.BlockSpec((_RB2, _B), lambda b: (b, 0)),
        ],
        out_specs=pl.BlockSpec((_RB2, _B), lambda b: (b, 0)),
        out_shape=jax.ShapeDtypeStruct((_B, _B), jnp.int32),
    )(cnt_s, ks, kn)


def _invert_body(rank_hbm, out_hbm, row_v, inv_v):
    wid = lax.axis_index("s") * _NC + lax.axis_index("c")
    rows_per = _B // _NW

    def row_step(r, carry):
        row = wid * rows_per + r
        pltpu.sync_copy(rank_hbm.at[row], row_v)

        def chunk(k, c2):
            idx = row_v[pl.ds(k * _L, _L)]
            vals = lax.broadcasted_iota(jnp.int32, (_L,), 0) + k * _L
            plsc.store_scatter(inv_v, [idx], vals)
            return c2

        lax.fori_loop(0, _B // _L, chunk, 0)
        pltpu.sync_copy(inv_v, out_hbm.at[row])
        return carry

    lax.fori_loop(0, rows_per, row_step, 0)


def _invert(rank):
    f = pl.kernel(
        _invert_body,
        mesh=plsc.VectorSubcoreMesh(core_axis_name="c", subcore_axis_name="s"),
        out_type=jax.ShapeDtypeStruct((_B, _B), jnp.int32),
        scratch_types=[
            pltpu.VMEM((_B,), jnp.int32),
            pltpu.VMEM((_B,), jnp.int32),
        ],
        compiler_params=pltpu.CompilerParams(needs_layout_passes=False),
    )
    return f(rank)


def _l2n(x):
    return x / jnp.maximum(jnp.linalg.norm(x, axis=1, keepdims=True), 1e-12)


def kernel(anchors, candidates, positive_mask):
    pm_f = positive_mask.astype(jnp.float32)
    ks, kn, loss_s, acc_s, cnt_s = _mine_stats(_l2n(anchors),
                                               _l2n(candidates), pm_f)
    rank = _rank(cnt_s, ks, kn)
    hard_indices = _invert(rank)
    loss = loss_s[0, 0] / _B
    accuracy = acc_s[0, 0] / _B
    return loss, accuracy, hard_indices


# speedup vs baseline: 1.2030x; 1.0010x over previous
import jax
import jax.numpy as jnp
from jax import lax
from jax.experimental import pallas as pl
from jax.experimental.pallas import tpu as pltpu
from jax.experimental.pallas import tpu_sc as plsc

_TEMPERATURE = 0.07
_MARGIN = 0.3
_B = 1024
_D = 64
_RB = 128
_RB2 = 32
_IB = 128
_NC = 2
_NS = 16
_NW = _NC * _NS
_L = 16


def _stats_body(a_ref, c_ref, pm_ref, ks_ref, kn_ref, loss_ref, acc_ref,
                cnt_ref):
    step = pl.program_id(0)
    an = a_ref[...]
    cn = c_ref[...]
    pm = pm_ref[...]
    sim = lax.dot_general(an, cn, (((1,), (1,)), ((), ())),
                          preferred_element_type=jnp.float32)
    cnt = jnp.sum(pm, axis=1, keepdims=True)
    pos = jnp.sum(sim * pm, axis=1, keepdims=True) / jnp.maximum(cnt, 1.0)
    negm = pm == 0.0
    semi = jnp.logical_and(sim > pos - _MARGIN, negm)
    neg_inf = jnp.float32(-jnp.inf)
    ks_ref[...] = jnp.where(semi, sim, neg_inf)
    kn_ref[...] = jnp.where(negm, sim, neg_inf)
    row_max = jnp.max(sim, axis=1, keepdims=True)
    m = jnp.maximum(row_max, pos)
    se = (jnp.sum(jnp.exp((sim - m) / _TEMPERATURE), axis=1, keepdims=True)
          + jnp.exp((pos - m) / _TEMPERATURE))
    loss_rows = m / _TEMPERATURE + jnp.log(se) - pos / _TEMPERATURE
    acc_rows = (pos >= row_max).astype(jnp.float32)
    lsum = jnp.sum(loss_rows)
    asum = jnp.sum(acc_rows)
    ssum = jnp.sum(semi.astype(jnp.float32))

    @pl.when(step == 0)
    def _():
        loss_ref[0, 0] = lsum
        acc_ref[0, 0] = asum
        cnt_ref[0, 0] = ssum

    @pl.when(step != 0)
    def _():
        loss_ref[0, 0] += lsum
        acc_ref[0, 0] += asum
        cnt_ref[0, 0] += ssum


def _mine_stats(anchors, candidates, pm_f):
    return pl.pallas_call(
        _stats_body,
        grid=(_B // _RB,),
        in_specs=[
            pl.BlockSpec((_RB, _D), lambda i: (i, 0)),
            pl.BlockSpec((_B, _D), lambda i: (0, 0)),
            pl.BlockSpec((_RB, _B), lambda i: (i, 0)),
        ],
        out_specs=[
            pl.BlockSpec((_RB, _B), lambda i: (i, 0)),
            pl.BlockSpec((_RB, _B), lambda i: (i, 0)),
            pl.BlockSpec(memory_space=pltpu.SMEM),
            pl.BlockSpec(memory_space=pltpu.SMEM),
            pl.BlockSpec(memory_space=pltpu.SMEM),
        ],
        out_shape=[
            jax.ShapeDtypeStruct((_B, _B), jnp.float32),
            jax.ShapeDtypeStruct((_B, _B), jnp.float32),
            jax.ShapeDtypeStruct((1, 1), jnp.float32),
            jax.ShapeDtypeStruct((1, 1), jnp.float32),
            jax.ShapeDtypeStruct((1, 1), jnp.float32),
        ],
    )(anchors, candidates, pm_f)


def _rank_body(cnt_ref, ksj_ref, knj_ref, out_ref):
    use_semi = cnt_ref[0, 0] > 0.0
    kj = jnp.where(use_semi, ksj_ref[...], knj_ref[...])
    one = jnp.float32(1.0)
    zero = jnp.float32(0.0)
    ii = lax.broadcasted_iota(jnp.int32, (_IB, _IB), 0)
    jj = lax.broadcasted_iota(jnp.int32, (_IB, _IB), 1)
    tie_f = jnp.where(ii < jj, one, zero)[None, :, :]
    tot = None
    for ib in range(_B // _IB):
        lo = ib * _IB
        hi = lo + _IB
        ki3 = kj[:, lo:hi, None]
        pieces = []
        if lo > 0:
            left = kj[:, None, :lo]
            pieces.append(jnp.sum(jnp.where(ki3 > left, one, zero), axis=1))
        diag = kj[:, None, lo:hi]
        d = (jnp.where(ki3 > diag, one, zero)
             + tie_f * jnp.where(ki3 == diag, one, zero))
        pieces.append(jnp.sum(d, axis=1))
        if hi < _B:
            right = kj[:, None, hi:]
            pieces.append(jnp.sum(jnp.where(ki3 >= right, one, zero), axis=1))
        contrib = jnp.concatenate(pieces, axis=1)
        tot = contrib if tot is None else tot + contrib
    out_ref[...] = tot.astype(jnp.int32)


def _rank(cnt_s, ks, kn):
    return pl.pallas_call(
        _rank_body,
        grid=(_B // _RB2,),
        in_specs=[
            pl.BlockSpec(memory_space=pltpu.SMEM),
            pl.BlockSpec((_RB2, _B), lambda b: (b, 0)),
            pl.BlockSpec((_RB2, _B), lambda b: (b, 0)),
        ],
        out_specs=pl.BlockSpec((_RB2, _B), lambda b: (b, 0)),
        out_shape=jax.ShapeDtypeStruct((_B, _B), jnp.int32),
    )(cnt_s, ks, kn)


def _invert_body(rank_hbm, out_hbm, row_v, inv_v):
    wid = lax.axis_index("s") * _NC + lax.axis_index("c")
    rows_per = _B // _NW

    def row_step(r, carry):
        row = wid * rows_per + r
        pltpu.sync_copy(rank_hbm.at[row], row_v)

        def chunk(k, c2):
            idx = row_v[pl.ds(k * _L, _L)]
            vals = lax.broadcasted_iota(jnp.int32, (_L,), 0) + k * _L
            plsc.store_scatter(inv_v, [idx], vals)
            return c2

        lax.fori_loop(0, _B // _L, chunk, 0)
        pltpu.sync_copy(inv_v, out_hbm.at[row])
        return carry

    lax.fori_loop(0, rows_per, row_step, 0)


def _invert(rank):
    f = pl.kernel(
        _invert_body,
        mesh=plsc.VectorSubcoreMesh(core_axis_name="c", subcore_axis_name="s"),
        out_type=jax.ShapeDtypeStruct((_B, _B), jnp.int32),
        scratch_types=[
            pltpu.VMEM((_B,), jnp.int32),
            pltpu.VMEM((_B,), jnp.int32),
        ],
        compiler_params=pltpu.CompilerParams(needs_layout_passes=False),
    )
    return f(rank)


def _l2n(x):
    return x / jnp.maximum(jnp.linalg.norm(x, axis=1, keepdims=True), 1e-12)


def kernel(anchors, candidates, positive_mask):
    pm_f = positive_mask.astype(jnp.float32)
    ks, kn, loss_s, acc_s, cnt_s = _mine_stats(_l2n(anchors),
                                               _l2n(candidates), pm_f)
    rank = _rank(cnt_s, ks, kn)
    hard_indices = _invert(rank)
    loss = loss_s[0, 0] / _B
    accuracy = acc_s[0, 0] / _B
    return loss, accuracy, hard_indices
